# Initial kernel scaffold; baseline (speedup 1.0000x reference)
import jax
import jax.numpy as jnp
from jax.experimental import pallas as pl


def kernel(x, edge_attr, edge_index, shapes, emb_params, block_params):
    raise NotImplementedError("write your pallas kernel here")


# speedup vs baseline: 2.4503x; 2.4503x over previous
import functools

import jax
import jax.numpy as jnp
from jax import lax
from jax.experimental import pallas as pl
from jax.experimental.pallas import tpu as pltpu
from jax.experimental.pallas import tpu_sc as plsc

F32 = jnp.float32


def _layernorm(h, g, b):
    mu = jnp.mean(h, axis=-1, keepdims=True)
    var = jnp.mean((h - mu) ** 2, axis=-1, keepdims=True)
    return (h - mu) * lax.rsqrt(var + 1e-5) * g + b


def _emb_body(ea_ref, w1_ref, b1_ref, w2_ref, b2_ref, g_ref, be_ref, o_ref):
    h = jnp.dot(ea_ref[...], w1_ref[...], preferred_element_type=F32) + b1_ref[...]
    h = h * jax.nn.sigmoid(h)
    h = jnp.dot(h, w2_ref[...], preferred_element_type=F32) + b2_ref[...]
    o_ref[...] = _layernorm(h, g_ref[...], be_ref[...])


def _emb_mlp(ea, p):
    n, d_in = ea.shape
    d = p['w2'].shape[1]
    blk = 2000
    grid = n // blk
    return pl.pallas_call(
        _emb_body,
        grid=(grid,),
        in_specs=[
            pl.BlockSpec((blk, d_in), lambda i: (i, 0)),
            pl.BlockSpec((d_in, d), lambda i: (0, 0)),
            pl.BlockSpec((1, d), lambda i: (0, 0)),
            pl.BlockSpec((d, d), lambda i: (0, 0)),
            pl.BlockSpec((1, d), lambda i: (0, 0)),
            pl.BlockSpec((1, d), lambda i: (0, 0)),
            pl.BlockSpec((1, d), lambda i: (0, 0)),
        ],
        out_specs=pl.BlockSpec((blk, d), lambda i: (i, 0)),
        out_shape=jax.ShapeDtypeStruct((n, d), F32),
    )(ea, p['w1'], p['b1'].reshape(1, d), p['w2'], p['b2'].reshape(1, d),
      p['ln_g'].reshape(1, d), p['ln_b'].reshape(1, d))


def _pair_linear_body(x_ref, wi_ref, wj_ref, yd_ref, ys_ref):
    x = x_ref[...]
    yd_ref[...] = jnp.dot(x, wi_ref[...], preferred_element_type=F32)
    ys_ref[...] = jnp.dot(x, wj_ref[...], preferred_element_type=F32)


def _pair_linear(x, wi, wj):
    n, d = x.shape
    blk = 2000
    return pl.pallas_call(
        _pair_linear_body,
        grid=(n // blk,),
        in_specs=[
            pl.BlockSpec((blk, d), lambda i: (i, 0)),
            pl.BlockSpec((d, d), lambda i: (0, 0)),
            pl.BlockSpec((d, d), lambda i: (0, 0)),
        ],
        out_specs=[
            pl.BlockSpec((blk, d), lambda i: (i, 0)),
            pl.BlockSpec((blk, d), lambda i: (i, 0)),
        ],
        out_shape=[
            jax.ShapeDtypeStruct((n, d), F32),
            jax.ShapeDtypeStruct((n, d), F32),
        ],
    )(x, wi, wj)


def _edge_body(gd_ref, gs_ref, ea_ref, w1e_ref, b1_ref, w2_ref, b2_ref,
               g_ref, be_ref, o_ref):
    ea = ea_ref[...]
    h = gd_ref[...] + gs_ref[...] + b1_ref[...]
    h = h + jnp.dot(ea, w1e_ref[...], preferred_element_type=F32)
    h = h * jax.nn.sigmoid(h)
    h = jnp.dot(h, w2_ref[...], preferred_element_type=F32) + b2_ref[...]
    o_ref[...] = _layernorm(h, g_ref[...], be_ref[...]) + ea


def _edge_mlp(gd, gs, ea, p):
    n, d = ea.shape
    w1e = p['w1'][2 * d:, :]
    blk = 2000
    return pl.pallas_call(
        _edge_body,
        grid=(n // blk,),
        in_specs=[
            pl.BlockSpec((blk, d), lambda i: (i, 0)),
            pl.BlockSpec((blk, d), lambda i: (i, 0)),
            pl.BlockSpec((blk, d), lambda i: (i, 0)),
            pl.BlockSpec((d, d), lambda i: (0, 0)),
            pl.BlockSpec((1, d), lambda i: (0, 0)),
            pl.BlockSpec((d, d), lambda i: (0, 0)),
            pl.BlockSpec((1, d), lambda i: (0, 0)),
            pl.BlockSpec((1, d), lambda i: (0, 0)),
            pl.BlockSpec((1, d), lambda i: (0, 0)),
        ],
        out_specs=pl.BlockSpec((blk, d), lambda i: (i, 0)),
        out_shape=jax.ShapeDtypeStruct((n, d), F32),
    )(gd, gs, ea, w1e, p['b1'].reshape(1, d), p['w2'], p['b2'].reshape(1, d),
      p['ln_g'].reshape(1, d), p['ln_b'].reshape(1, d))


def _node_body(x_ref, p0_ref, p1_ref, v1x_ref, v1a_ref, c1_ref, v2_ref,
               c2_ref, g_ref, be_ref, o_ref):
    x = x_ref[...]
    agg = p0_ref[...] + p1_ref[...]
    h = jnp.dot(x, v1x_ref[...], preferred_element_type=F32) + c1_ref[...]
    h = h + jnp.dot(agg, v1a_ref[...], preferred_element_type=F32)
    h = h * jax.nn.sigmoid(h)
    h = jnp.dot(h, v2_ref[...], preferred_element_type=F32) + c2_ref[...]
    o_ref[...] = _layernorm(h, g_ref[...], be_ref[...]) + x


def _node_mlp(x, p0, p1, p):
    n, d = x.shape
    v1x = p['w1'][:d, :]
    v1a = p['w1'][d:, :]
    blk = 2000
    return pl.pallas_call(
        _node_body,
        grid=(n // blk,),
        in_specs=[
            pl.BlockSpec((blk, d), lambda i: (i, 0)),
            pl.BlockSpec((blk, d), lambda i: (i, 0)),
            pl.BlockSpec((blk, d), lambda i: (i, 0)),
            pl.BlockSpec((d, d), lambda i: (0, 0)),
            pl.BlockSpec((d, d), lambda i: (0, 0)),
            pl.BlockSpec((1, d), lambda i: (0, 0)),
            pl.BlockSpec((d, d), lambda i: (0, 0)),
            pl.BlockSpec((1, d), lambda i: (0, 0)),
            pl.BlockSpec((1, d), lambda i: (0, 0)),
            pl.BlockSpec((1, d), lambda i: (0, 0)),
        ],
        out_specs=pl.BlockSpec((blk, d), lambda i: (i, 0)),
        out_shape=jax.ShapeDtypeStruct((n, d), F32),
    )(x, p0, p1, v1x, v1a, p['b1'].reshape(1, d), p['w2'],
      p['b2'].reshape(1, d), p['ln_g'].reshape(1, d), p['ln_b'].reshape(1, d))


_NC = 2
_NS = 16
_NW = _NC * _NS
_CHUNK = 80


def _sc_gather(table, idx):
    n_edges = idx.shape[0]
    n, d = table.shape
    per_w = n_edges // _NW
    mesh = plsc.VectorSubcoreMesh(core_axis_name="c", subcore_axis_name="s")

    @functools.partial(
        pl.kernel, mesh=mesh,
        out_type=jax.ShapeDtypeStruct((n_edges, d), F32),
        scratch_types=[
            pltpu.VMEM((_CHUNK,), jnp.int32),
            pltpu.VMEM((_CHUNK, d), F32),
            pltpu.SemaphoreType.DMA,
        ],
    )
    def k(table_hbm, idx_hbm, out_hbm, idx_v, rows_v, sem):
        wid = lax.axis_index("s") * _NC + lax.axis_index("c")
        base = wid * per_w

        @pl.loop(0, per_w, step=_CHUNK)
        def _(i):
            pltpu.sync_copy(idx_hbm.at[pl.ds(base + i, _CHUNK)], idx_v)
            pltpu.async_copy(table_hbm.at[idx_v], rows_v, sem).wait()
            pltpu.sync_copy(rows_v, out_hbm.at[pl.ds(base + i, _CHUNK)])

    return k(table, idx)


def _sc_segsum(msgs, idx, zeros):
    n_edges, d = msgs.shape
    n = zeros.shape[0]
    per_w = n_edges // _NW
    rows_per_sub = (n // _NS) // 8 * 8
    tail_start = rows_per_sub * _NS
    tail = n - tail_start
    mesh = plsc.VectorSubcoreMesh(core_axis_name="c", subcore_axis_name="s")

    @functools.partial(
        pl.kernel, mesh=mesh,
        out_type=jax.ShapeDtypeStruct((_NC, n, d), F32),
        scratch_types=[
            pltpu.VMEM((_CHUNK,), jnp.int32),
            pltpu.VMEM((_CHUNK, d), F32),
            pltpu.VMEM_SHARED((n, d), F32),
            pltpu.SemaphoreType.DMA,
        ],
    )
    def k(msgs_hbm, idx_hbm, zeros_hbm, out_hbm, idx_v, rows_v, agg_sh, sem):
        cid = lax.axis_index("c")
        sid = lax.axis_index("s")
        wid = sid * _NC + cid
        r0 = sid * rows_per_sub
        pltpu.sync_copy(zeros_hbm.at[pl.ds(r0, rows_per_sub)],
                        agg_sh.at[pl.ds(r0, rows_per_sub)])

        @pl.when(sid == 0)
        def _():
            pltpu.sync_copy(zeros_hbm.at[pl.ds(tail_start, tail)],
                            agg_sh.at[pl.ds(tail_start, tail)])

        plsc.subcore_barrier()
        base = wid * per_w

        @pl.loop(0, per_w, step=_CHUNK)
        def _(i):
            pltpu.sync_copy(idx_hbm.at[pl.ds(base + i, _CHUNK)], idx_v)
            pltpu.sync_copy(msgs_hbm.at[pl.ds(base + i, _CHUNK)], rows_v)
            pltpu.sync_copy(rows_v, agg_sh.at[idx_v], add=True)

        plsc.subcore_barrier()
        pltpu.sync_copy(agg_sh.at[pl.ds(r0, rows_per_sub)],
                        out_hbm.at[cid].at[pl.ds(r0, rows_per_sub)])

        @pl.when(sid == 0)
        def _():
            pltpu.sync_copy(agg_sh.at[pl.ds(tail_start, tail)],
                            out_hbm.at[cid].at[pl.ds(tail_start, tail)])

    return k(msgs, idx, zeros)


def kernel(x, edge_attr, edge_index, shapes, emb_params, block_params):
    del shapes
    n, d = x.shape
    src = edge_index[0]
    dst = edge_index[1]
    zeros = jnp.zeros((n, d), F32)

    ea = _emb_mlp(edge_attr, emb_params)
    x_out = x
    for p in block_params:
        w1 = p['edge_mlp']['w1']
        yd, ys = _pair_linear(x_out, w1[:d, :], w1[d:2 * d, :])
        gd = _sc_gather(yd, dst)
        gs = _sc_gather(ys, src)
        en = _edge_mlp(gd, gs, ea, p['edge_mlp'])
        parts = _sc_segsum(en, dst, zeros)
        x_out = _node_mlp(x_out, parts[0], parts[1], p['node_mlp'])
        ea = en
    return (x_out, ea)


# speedup vs baseline: 4.1621x; 1.6986x over previous
import functools

import jax
import jax.numpy as jnp
from jax import lax
from jax.experimental import pallas as pl
from jax.experimental.pallas import tpu as pltpu
from jax.experimental.pallas import tpu_sc as plsc

F32 = jnp.float32


def _layernorm(h, g, b):
    mu = jnp.mean(h, axis=-1, keepdims=True)
    var = jnp.mean((h - mu) ** 2, axis=-1, keepdims=True)
    return (h - mu) * lax.rsqrt(var + 1e-5) * g + b


def _emb_body(ea_ref, w1_ref, b1_ref, w2_ref, b2_ref, g_ref, be_ref, o_ref):
    h = jnp.dot(ea_ref[...], w1_ref[...], preferred_element_type=F32) + b1_ref[...]
    h = h * jax.nn.sigmoid(h)
    h = jnp.dot(h, w2_ref[...], preferred_element_type=F32) + b2_ref[...]
    o_ref[...] = _layernorm(h, g_ref[...], be_ref[...])


def _emb_mlp(ea, p):
    n, d_in = ea.shape
    d = p['w2'].shape[1]
    blk = 2000
    grid = n // blk
    return pl.pallas_call(
        _emb_body,
        grid=(grid,),
        in_specs=[
            pl.BlockSpec((blk, d_in), lambda i: (i, 0)),
            pl.BlockSpec((d_in, d), lambda i: (0, 0)),
            pl.BlockSpec((1, d), lambda i: (0, 0)),
            pl.BlockSpec((d, d), lambda i: (0, 0)),
            pl.BlockSpec((1, d), lambda i: (0, 0)),
            pl.BlockSpec((1, d), lambda i: (0, 0)),
            pl.BlockSpec((1, d), lambda i: (0, 0)),
        ],
        out_specs=pl.BlockSpec((blk, d), lambda i: (i, 0)),
        out_shape=jax.ShapeDtypeStruct((n, d), F32),
    )(ea, p['w1'], p['b1'].reshape(1, d), p['w2'], p['b2'].reshape(1, d),
      p['ln_g'].reshape(1, d), p['ln_b'].reshape(1, d))


def _pair_linear_body(x_ref, wi_ref, wj_ref, yd_ref, ys_ref):
    x = x_ref[...]
    yd_ref[...] = jnp.dot(x, wi_ref[...], preferred_element_type=F32)
    ys_ref[...] = jnp.dot(x, wj_ref[...], preferred_element_type=F32)


def _pair_linear(x, wi, wj):
    n, d = x.shape
    blk = 2000
    return pl.pallas_call(
        _pair_linear_body,
        grid=(n // blk,),
        in_specs=[
            pl.BlockSpec((blk, d), lambda i: (i, 0)),
            pl.BlockSpec((d, d), lambda i: (0, 0)),
            pl.BlockSpec((d, d), lambda i: (0, 0)),
        ],
        out_specs=[
            pl.BlockSpec((blk, d), lambda i: (i, 0)),
            pl.BlockSpec((blk, d), lambda i: (i, 0)),
        ],
        out_shape=[
            jax.ShapeDtypeStruct((n, d), F32),
            jax.ShapeDtypeStruct((n, d), F32),
        ],
    )(x, wi, wj)


def _edge_body(gd_ref, gs_ref, ea_ref, w1e_ref, b1_ref, w2_ref, b2_ref,
               g_ref, be_ref, o_ref):
    ea = ea_ref[...]
    h = gd_ref[...].astype(F32) + gs_ref[...].astype(F32) + b1_ref[...]
    h = h + jnp.dot(ea, w1e_ref[...], preferred_element_type=F32)
    h = h * jax.nn.sigmoid(h)
    h = jnp.dot(h, w2_ref[...], preferred_element_type=F32) + b2_ref[...]
    o_ref[...] = _layernorm(h, g_ref[...], be_ref[...]) + ea


def _edge_mlp(gd, gs, ea, p):
    n, d = ea.shape
    w1e = p['w1'][2 * d:, :]
    blk = 2000
    return pl.pallas_call(
        _edge_body,
        grid=(n // blk,),
        in_specs=[
            pl.BlockSpec((blk, d), lambda i: (i, 0)),
            pl.BlockSpec((blk, d), lambda i: (i, 0)),
            pl.BlockSpec((blk, d), lambda i: (i, 0)),
            pl.BlockSpec((d, d), lambda i: (0, 0)),
            pl.BlockSpec((1, d), lambda i: (0, 0)),
            pl.BlockSpec((d, d), lambda i: (0, 0)),
            pl.BlockSpec((1, d), lambda i: (0, 0)),
            pl.BlockSpec((1, d), lambda i: (0, 0)),
            pl.BlockSpec((1, d), lambda i: (0, 0)),
        ],
        out_specs=pl.BlockSpec((blk, d), lambda i: (i, 0)),
        out_shape=jax.ShapeDtypeStruct((n, d), F32),
    )(gd, gs, ea, w1e, p['b1'].reshape(1, d), p['w2'], p['b2'].reshape(1, d),
      p['ln_g'].reshape(1, d), p['ln_b'].reshape(1, d))


def _node_body(x_ref, p0_ref, p1_ref, v1x_ref, v1a_ref, c1_ref, v2_ref,
               c2_ref, g_ref, be_ref, o_ref):
    x = x_ref[...]
    agg = p0_ref[...] + p1_ref[...]
    h = jnp.dot(x, v1x_ref[...], preferred_element_type=F32) + c1_ref[...]
    h = h + jnp.dot(agg, v1a_ref[...], preferred_element_type=F32)
    h = h * jax.nn.sigmoid(h)
    h = jnp.dot(h, v2_ref[...], preferred_element_type=F32) + c2_ref[...]
    o_ref[...] = _layernorm(h, g_ref[...], be_ref[...]) + x


def _node_mlp(x, p0, p1, p):
    n, d = x.shape
    v1x = p['w1'][:d, :]
    v1a = p['w1'][d:, :]
    blk = 2000
    return pl.pallas_call(
        _node_body,
        grid=(n // blk,),
        in_specs=[
            pl.BlockSpec((blk, d), lambda i: (i, 0)),
            pl.BlockSpec((blk, d), lambda i: (i, 0)),
            pl.BlockSpec((blk, d), lambda i: (i, 0)),
            pl.BlockSpec((d, d), lambda i: (0, 0)),
            pl.BlockSpec((d, d), lambda i: (0, 0)),
            pl.BlockSpec((1, d), lambda i: (0, 0)),
            pl.BlockSpec((d, d), lambda i: (0, 0)),
            pl.BlockSpec((1, d), lambda i: (0, 0)),
            pl.BlockSpec((1, d), lambda i: (0, 0)),
            pl.BlockSpec((1, d), lambda i: (0, 0)),
        ],
        out_specs=pl.BlockSpec((blk, d), lambda i: (i, 0)),
        out_shape=jax.ShapeDtypeStruct((n, d), F32),
    )(x, p0, p1, v1x, v1a, p['b1'].reshape(1, d), p['w2'],
      p['b2'].reshape(1, d), p['ln_g'].reshape(1, d), p['ln_b'].reshape(1, d))


_NC = 2
_NS = 16
_NW = _NC * _NS
_W = 128


def _sc_gather_spmem(table, idx2d):
    n, d = table.shape
    n_edges = idx2d.shape[1]
    nblk = n_edges // _W
    main = (nblk // _NW) * _NW
    tail_blocks = nblk - main
    rows_per_sub = (n // _NS) // 8 * 8
    tail_start = rows_per_sub * _NS
    tail_rows = n - tail_start
    mesh = plsc.VectorSubcoreMesh(core_axis_name="c", subcore_axis_name="s")

    @functools.partial(
        pl.kernel, mesh=mesh,
        out_type=jax.ShapeDtypeStruct((n_edges, d), F32),
        scratch_types=[
            pltpu.VMEM_SHARED((n, d), F32),
            pltpu.VMEM((_W,), jnp.int32),
            pltpu.VMEM((_W, d), F32),
        ],
    )
    def k(tbl_hbm, di_hbm, out_hbm, tbl_sh, idx_tv, rows_tv):
        cid = lax.axis_index("c")
        sid = lax.axis_index("s")
        r0 = sid * rows_per_sub
        pltpu.sync_copy(tbl_hbm.at[pl.ds(r0, rows_per_sub)],
                        tbl_sh.at[pl.ds(r0, rows_per_sub)])

        @pl.when(sid == 0)
        def _():
            pltpu.sync_copy(tbl_hbm.at[pl.ds(tail_start, tail_rows)],
                            tbl_sh.at[pl.ds(tail_start, tail_rows)])

        plsc.subcore_barrier()

        def body(di_v, o_v):
            pltpu.sync_copy(tbl_sh.at[di_v.at[0]], o_v)

        pltpu.emit_pipeline(
            body,
            grid=(main,),
            in_specs=[pl.BlockSpec((1, _W), lambda i: (0, i))],
            out_specs=[pl.BlockSpec((_W, d), lambda i: (i, 0))],
            core_axis_name=("c", "s"),
            dimension_semantics=(pltpu.PARALLEL,),
        )(di_hbm, out_hbm)

        wid = sid * _NC + cid

        @pl.when(wid < tail_blocks)
        def _():
            base = (main + wid) * _W
            pltpu.sync_copy(di_hbm.at[0].at[pl.ds(base, _W)], idx_tv)
            pltpu.sync_copy(tbl_sh.at[idx_tv], rows_tv)
            pltpu.sync_copy(rows_tv, out_hbm.at[pl.ds(base, _W)])

    return k(table, idx2d)


def _sc_segsum(msgs, idx2d, zeros):
    n_edges, d = msgs.shape
    n = zeros.shape[0]
    nblk = n_edges // _W
    main = (nblk // _NW) * _NW
    tail_blocks = nblk - main
    rows_per_sub = (n // _NS) // 8 * 8
    tail_start = rows_per_sub * _NS
    tail = n - tail_start
    mesh = plsc.VectorSubcoreMesh(core_axis_name="c", subcore_axis_name="s")

    @functools.partial(
        pl.kernel, mesh=mesh,
        out_type=jax.ShapeDtypeStruct((_NC, n, d), F32),
        scratch_types=[
            pltpu.VMEM((_W,), jnp.int32),
            pltpu.VMEM((_W, d), F32),
            pltpu.VMEM_SHARED((n, d), F32),
        ],
    )
    def k(msgs_hbm, idx_hbm, zeros_hbm, out_hbm, idx_v, rows_v, agg_sh):
        cid = lax.axis_index("c")
        sid = lax.axis_index("s")
        wid = sid * _NC + cid
        r0 = sid * rows_per_sub
        pltpu.sync_copy(zeros_hbm.at[pl.ds(r0, rows_per_sub)],
                        agg_sh.at[pl.ds(r0, rows_per_sub)])

        @pl.when(sid == 0)
        def _():
            pltpu.sync_copy(zeros_hbm.at[pl.ds(tail_start, tail)],
                            agg_sh.at[pl.ds(tail_start, tail)])

        plsc.subcore_barrier()

        def body(m_v, di_v):
            pltpu.sync_copy(m_v, agg_sh.at[di_v.at[0]], add=True)

        pltpu.emit_pipeline(
            body,
            grid=(main,),
            in_specs=[
                pl.BlockSpec((_W, d), lambda i: (i, 0)),
                pl.BlockSpec((1, _W), lambda i: (0, i)),
            ],
            out_specs=[],
            core_axis_name=("c", "s"),
            dimension_semantics=(pltpu.PARALLEL,),
        )(msgs_hbm, idx_hbm)

        @pl.when(wid < tail_blocks)
        def _():
            base = (main + wid) * _W
            pltpu.sync_copy(idx_hbm.at[0].at[pl.ds(base, _W)], idx_v)
            pltpu.sync_copy(msgs_hbm.at[pl.ds(base, _W)], rows_v)
            pltpu.sync_copy(rows_v, agg_sh.at[idx_v], add=True)

        plsc.subcore_barrier()
        pltpu.sync_copy(agg_sh.at[pl.ds(r0, rows_per_sub)],
                        out_hbm.at[cid].at[pl.ds(r0, rows_per_sub)])

        @pl.when(sid == 0)
        def _():
            pltpu.sync_copy(agg_sh.at[pl.ds(tail_start, tail)],
                            out_hbm.at[cid].at[pl.ds(tail_start, tail)])

    return k(msgs, idx2d, zeros)


def kernel(x, edge_attr, edge_index, shapes, emb_params, block_params):
    del shapes
    n, d = x.shape
    src2d = edge_index[0].reshape(1, -1)
    dst2d = edge_index[1].reshape(1, -1)
    zeros = jnp.zeros((n, d), F32)

    ea = _emb_mlp(edge_attr, emb_params)
    x_out = x
    for p in block_params:
        w1 = p['edge_mlp']['w1']
        yd, ys = _pair_linear(x_out, w1[:d, :], w1[d:2 * d, :])
        gd = _sc_gather_spmem(yd, dst2d)
        gs = _sc_gather_spmem(ys, src2d)
        en = _edge_mlp(gd, gs, ea, p['edge_mlp'])
        parts = _sc_segsum(en, dst2d, zeros)
        x_out = _node_mlp(x_out, parts[0], parts[1], p['node_mlp'])
        ea = en
    return (x_out, ea)
